# baseline (speedup 1.0000x reference)
import functools

import jax
import jax.numpy as jnp
from jax import lax
from jax.experimental import pallas as pl
from jax.experimental.pallas import tpu as pltpu
from jax.experimental.pallas import tpu_sc as plsc

B = 4096
KNN = 32
TARGET = -0.1
RB = 128
QR = 16
NBLK = B // RB
NW = 32
RPW = B // NW
CH = 512


def _topk_body(d_ref, idx_ref, x_ref):
    inf = jnp.float32(jnp.inf)
    bigi = jnp.int32(1 << 30)
    lane = lax.broadcasted_iota(jnp.int32, (QR, 128), 1)
    for q in range(RB // QR):
        r0 = q * QR
        am = None
        for t in range(KNN + 1):
            aml = None if t == 0 else am - lane
            vmin = None
            vgrp = jnp.zeros((QR, 128), jnp.int32)
            for g in range(32):
                src = d_ref if t <= 1 else x_ref
                xg = src[r0:r0 + QR, g * 128:(g + 1) * 128]
                if t >= 1:
                    xg = jnp.where(aml == g * 128, inf, xg)
                    x_ref[r0:r0 + QR, g * 128:(g + 1) * 128] = xg
                if g == 0:
                    vmin = xg
                else:
                    lt = xg < vmin
                    vgrp = jnp.where(lt, jnp.int32(g), vgrp)
                    vmin = jnp.minimum(xg, vmin)
            m = jnp.min(vmin, axis=1, keepdims=True)
            cand = jnp.where(vmin == m, vgrp * 128 + lane, bigi)
            am = jnp.min(cand, axis=1, keepdims=True)
            if t >= 1:
                idx_ref[t - 1, r0:r0 + QR] = am[:, 0]


def _topk(D):
    return pl.pallas_call(
        _topk_body,
        grid=(NBLK,),
        in_specs=[pl.BlockSpec((RB, B), lambda g: (g, 0))],
        out_specs=pl.BlockSpec((KNN, RB), lambda g: (0, g)),
        out_shape=jax.ShapeDtypeStruct((KNN, B), jnp.int32),
        scratch_shapes=[pltpu.VMEM((RB, B), jnp.float32)],
    )(D)


NWORD = B // 32
BMW = B * NWORD
RPS = B // 16
NCHUNK = RPS * KNN // 128


def _phase2_body(idxT_hbm, rows_hbm, out_hbm,
                 myidx, cbuf, hidx, widx, bval, ones1, zbuf,
                 qbits, hist, deg, outv, bitmap_sh, hist_sh):
    cid = lax.axis_index("c")
    sid = lax.axis_index("s")
    wid = cid * 16 + sid
    base = wid * RPW
    iota16 = lax.iota(jnp.int32, 16)
    zeros16 = jnp.zeros((16,), jnp.int32)
    ones16 = jnp.ones((16,), jnp.int32)

    pltpu.sync_copy(idxT_hbm.at[:, pl.ds(base, RPW)], myidx)
    pltpu.sync_copy(rows_hbm.at[pl.ds(sid * RPS, RPS)], cbuf)

    def zb(i, _):
        zbuf[pl.ds(i * 16, 16)] = zeros16
        return 0
    lax.fori_loop(0, 4096 // 16, zb, 0)

    for h in range(8):
        ones1[pl.ds(h * 16, 16)] = ones16
    for q in range(8):
        pltpu.sync_copy(zbuf, bitmap_sh.at[pl.ds(sid * 32768 + q * 4096, 4096)])
    pltpu.sync_copy(zbuf.at[pl.ds(0, B // 16)],
                    hist_sh.at[pl.ds(sid * (B // 16), B // 16)])

    def pay(g, _):
        for rr in range(4):
            r = g * 4 + rr
            i_val = sid * RPS + r
            for h in range(2):
                jv = cbuf[r, pl.ds(h * 16, 16)]
                col = rr * 32 + h * 16
                hidx[g, pl.ds(col, 16)] = jv
                widx[g, pl.ds(col, 16)] = (
                    lax.shift_right_logical(jv, 5) * B + i_val)
                bval[g, pl.ds(col, 16)] = lax.shift_left(ones16, jv & 31)
        return 0
    lax.fori_loop(0, NCHUNK, pay, 0)
    plsc.subcore_barrier()

    def scat(g, _):
        pltpu.sync_copy(ones1, hist_sh.at[hidx.at[g]], add=True)
        pltpu.sync_copy(bval.at[g], bitmap_sh.at[widx.at[g]], add=True)
        return 0
    lax.fori_loop(0, NCHUNK, scat, 0)
    plsc.subcore_barrier()

    pltpu.sync_copy(hist_sh, hist)
    pltpu.sync_copy(bitmap_sh.at[pl.ds(wid * (4 * B), 4 * B)], qbits)

    def dbody(i, _):
        iv = hist[pl.ds(i * 16, 16)]
        deg[pl.ds(i * 16, 16)] = (
            jnp.float32(KNN) + iv.astype(jnp.float32)) * 0.5
        return 0
    lax.fori_loop(0, B // 16, dbody, 0)

    def sbody(t, carry):
        muts, nmaxs = carry
        new_muts, new_nmaxs = [], []
        for c in range(8):
            jv = myidx[t, pl.ds(c * 16, 16)]
            dv = plsc.load_gather(deg, [jv])
            q = plsc.load_gather(qbits, [jv + (c // 2) * B])
            bit = lax.shift_right_logical(q, (c % 2) * 16 + iota16) & 1
            new_muts.append(muts[c] + bit)
            new_nmaxs.append(jnp.maximum(nmaxs[c], dv))
        return tuple(new_muts), tuple(new_nmaxs)

    z8 = tuple(jnp.zeros((16,), jnp.int32) for _ in range(8))
    n8 = tuple(jnp.full((16,), -jnp.inf, jnp.float32) for _ in range(8))
    muts, nmaxs = lax.fori_loop(0, KNN, sbody, (z8, n8))

    s_acc = jnp.zeros((16,), jnp.float32)
    cnt_acc = jnp.zeros((16,), jnp.float32)
    mx_acc = jnp.full((16,), -jnp.inf, jnp.float32)
    for c in range(8):
        iv = plsc.load_gather(hist, [base + c * 16 + iota16])
        indeg = iv.astype(jnp.float32)
        degc = (jnp.float32(KNN) + indeg) * 0.5
        edeg = jnp.float32(KNN) + indeg - muts[c].astype(jnp.float32)
        s_acc = s_acc + degc * edeg
        cnt_acc = cnt_acc + edeg
        mx_acc = jnp.maximum(mx_acc, degc + nmaxs[c])
    outv[0, :] = s_acc
    outv[1, :] = cnt_acc
    outv[2, :] = mx_acc
    pltpu.sync_copy(outv, out_hbm.at[wid])


@functools.cache
def _phase2_sc_kernel():
    return pl.kernel(
        _phase2_body,
        out_type=jax.ShapeDtypeStruct((NW, 3, 16), jnp.float32),
        mesh=plsc.VectorSubcoreMesh(core_axis_name="c", subcore_axis_name="s"),
        compiler_params=pltpu.CompilerParams(needs_layout_passes=False),
        scratch_types=[
            pltpu.VMEM((KNN, RPW), jnp.int32),
            pltpu.VMEM((RPS, KNN), jnp.int32),
            pltpu.VMEM((NCHUNK, 128), jnp.int32),
            pltpu.VMEM((NCHUNK, 128), jnp.int32),
            pltpu.VMEM((NCHUNK, 128), jnp.int32),
            pltpu.VMEM((128,), jnp.int32),
            pltpu.VMEM((4096,), jnp.int32),
            pltpu.VMEM((4 * B,), jnp.int32),
            pltpu.VMEM((B,), jnp.int32),
            pltpu.VMEM((B,), jnp.float32),
            pltpu.VMEM((3, 16), jnp.float32),
            pltpu.VMEM_SHARED((BMW,), jnp.int32),
            pltpu.VMEM_SHARED((B,), jnp.int32),
        ],
    )


def _phase2_jnp(idxT):
    idx = idxT.T
    indeg = jnp.zeros((B,), jnp.float32).at[idx.reshape(-1)].add(1.0)
    deg = (KNN + indeg) * 0.5
    g = idx[idx]
    mut = (g == jnp.arange(B)[:, None, None]).any(axis=2).sum(
        axis=1).astype(jnp.float32)
    edeg = KNN + indeg - mut
    cnt = edeg.sum()
    s = (deg * edeg).sum()
    kmean = 4.0 - 2.0 * s / cnt
    nmax = deg[idx].max(axis=1)
    kmin = 4.0 - (deg + nmax).max()
    loss = (kmean - TARGET) ** 2
    return loss, kmean, kmin


def kernel(D):
    idxT = _topk(D)
    parts = _phase2_sc_kernel()(idxT, idxT.T)
    s = parts[:, 0, :].sum()
    cnt = parts[:, 1, :].sum()
    mx = parts[:, 2, :].max()
    kmean = 4.0 - 2.0 * s / cnt
    kmin = 4.0 - mx
    loss = (kmean - TARGET) ** 2
    return loss, kmean, kmin


# speedup vs baseline: 1.0073x; 1.0073x over previous
import functools

import jax
import jax.numpy as jnp
from jax import lax
from jax.experimental import pallas as pl
from jax.experimental.pallas import tpu as pltpu
from jax.experimental.pallas import tpu_sc as plsc

B = 4096
KNN = 32
TARGET = -0.1
RB = 128
QR = 32
NBLK = B // RB
NW = 32
RPW = B // NW
CH = 512


def _topk_body(d_ref, idx_ref, x_ref):
    inf = jnp.float32(jnp.inf)
    bigi = jnp.int32(1 << 30)
    lane = lax.broadcasted_iota(jnp.int32, (QR, 128), 1)
    for q in range(RB // QR):
        r0 = q * QR
        am = None
        for t in range(KNN + 1):
            aml = None if t == 0 else am - lane
            vm = [None, None]
            vg = [jnp.zeros((QR, 128), jnp.int32) for _ in range(2)]
            for g in range(32):
                h = g // 16
                src = d_ref if t <= 1 else x_ref
                xg = src[r0:r0 + QR, g * 128:(g + 1) * 128]
                if t >= 1:
                    xg = jnp.where(aml == g * 128, inf, xg)
                    x_ref[r0:r0 + QR, g * 128:(g + 1) * 128] = xg
                if vm[h] is None:
                    vm[h] = xg
                    vg[h] = jnp.full((QR, 128), jnp.int32(g))
                else:
                    lt = xg < vm[h]
                    vg[h] = jnp.where(lt, jnp.int32(g), vg[h])
                    vm[h] = jnp.minimum(xg, vm[h])
            lt = vm[1] < vm[0]
            vgrp = jnp.where(lt, vg[1], vg[0])
            vmin = jnp.minimum(vm[0], vm[1])
            m = jnp.min(vmin, axis=1, keepdims=True)
            cand = jnp.where(vmin == m, vgrp * 128 + lane, bigi)
            am = jnp.min(cand, axis=1, keepdims=True)
            if t >= 1:
                idx_ref[t - 1, r0:r0 + QR] = am[:, 0]


def _topk(D):
    return pl.pallas_call(
        _topk_body,
        grid=(NBLK,),
        in_specs=[pl.BlockSpec((RB, B), lambda g: (g, 0))],
        out_specs=pl.BlockSpec((KNN, RB), lambda g: (0, g)),
        out_shape=jax.ShapeDtypeStruct((KNN, B), jnp.int32),
        scratch_shapes=[pltpu.VMEM((RB, B), jnp.float32)],
    )(D)


NWORD = B // 32
BMW = B * NWORD
RPS = B // 16
NCHUNK = RPS * KNN // 128


def _phase2_body(idxT_hbm, rows_hbm, out_hbm,
                 myidx, cbuf, hidx, widx, bval, ones1, zbuf,
                 qbits, hist, deg, outv, bitmap_sh, hist_sh):
    cid = lax.axis_index("c")
    sid = lax.axis_index("s")
    wid = cid * 16 + sid
    base = wid * RPW
    iota16 = lax.iota(jnp.int32, 16)
    zeros16 = jnp.zeros((16,), jnp.int32)
    ones16 = jnp.ones((16,), jnp.int32)

    pltpu.sync_copy(idxT_hbm.at[:, pl.ds(base, RPW)], myidx)
    pltpu.sync_copy(rows_hbm.at[pl.ds(sid * RPS, RPS)], cbuf)

    def zb(i, _):
        zbuf[pl.ds(i * 16, 16)] = zeros16
        return 0
    lax.fori_loop(0, 4096 // 16, zb, 0)

    for h in range(8):
        ones1[pl.ds(h * 16, 16)] = ones16
    for q in range(8):
        pltpu.sync_copy(zbuf, bitmap_sh.at[pl.ds(sid * 32768 + q * 4096, 4096)])
    pltpu.sync_copy(zbuf.at[pl.ds(0, B // 16)],
                    hist_sh.at[pl.ds(sid * (B // 16), B // 16)])

    def pay(g, _):
        for rr in range(4):
            r = g * 4 + rr
            i_val = sid * RPS + r
            for h in range(2):
                jv = cbuf[r, pl.ds(h * 16, 16)]
                col = rr * 32 + h * 16
                hidx[g, pl.ds(col, 16)] = jv
                widx[g, pl.ds(col, 16)] = (
                    lax.shift_right_logical(jv, 5) * B + i_val)
                bval[g, pl.ds(col, 16)] = lax.shift_left(ones16, jv & 31)
        return 0
    lax.fori_loop(0, NCHUNK, pay, 0)
    plsc.subcore_barrier()

    def scat(g, _):
        pltpu.sync_copy(ones1, hist_sh.at[hidx.at[g]], add=True)
        pltpu.sync_copy(bval.at[g], bitmap_sh.at[widx.at[g]], add=True)
        return 0
    lax.fori_loop(0, NCHUNK, scat, 0)
    plsc.subcore_barrier()

    pltpu.sync_copy(hist_sh, hist)
    pltpu.sync_copy(bitmap_sh.at[pl.ds(wid * (4 * B), 4 * B)], qbits)

    def dbody(i, _):
        iv = hist[pl.ds(i * 16, 16)]
        deg[pl.ds(i * 16, 16)] = (
            jnp.float32(KNN) + iv.astype(jnp.float32)) * 0.5
        return 0
    lax.fori_loop(0, B // 16, dbody, 0)

    def sbody(t, carry):
        muts, nmaxs = carry
        new_muts, new_nmaxs = [], []
        for c in range(8):
            jv = myidx[t, pl.ds(c * 16, 16)]
            dv = plsc.load_gather(deg, [jv])
            q = plsc.load_gather(qbits, [jv + (c // 2) * B])
            bit = lax.shift_right_logical(q, (c % 2) * 16 + iota16) & 1
            new_muts.append(muts[c] + bit)
            new_nmaxs.append(jnp.maximum(nmaxs[c], dv))
        return tuple(new_muts), tuple(new_nmaxs)

    z8 = tuple(jnp.zeros((16,), jnp.int32) for _ in range(8))
    n8 = tuple(jnp.full((16,), -jnp.inf, jnp.float32) for _ in range(8))
    muts, nmaxs = lax.fori_loop(0, KNN, sbody, (z8, n8))

    s_acc = jnp.zeros((16,), jnp.float32)
    cnt_acc = jnp.zeros((16,), jnp.float32)
    mx_acc = jnp.full((16,), -jnp.inf, jnp.float32)
    for c in range(8):
        iv = plsc.load_gather(hist, [base + c * 16 + iota16])
        indeg = iv.astype(jnp.float32)
        degc = (jnp.float32(KNN) + indeg) * 0.5
        edeg = jnp.float32(KNN) + indeg - muts[c].astype(jnp.float32)
        s_acc = s_acc + degc * edeg
        cnt_acc = cnt_acc + edeg
        mx_acc = jnp.maximum(mx_acc, degc + nmaxs[c])
    outv[0, :] = s_acc
    outv[1, :] = cnt_acc
    outv[2, :] = mx_acc
    pltpu.sync_copy(outv, out_hbm.at[wid])


@functools.cache
def _phase2_sc_kernel():
    return pl.kernel(
        _phase2_body,
        out_type=jax.ShapeDtypeStruct((NW, 3, 16), jnp.float32),
        mesh=plsc.VectorSubcoreMesh(core_axis_name="c", subcore_axis_name="s"),
        compiler_params=pltpu.CompilerParams(needs_layout_passes=False),
        scratch_types=[
            pltpu.VMEM((KNN, RPW), jnp.int32),
            pltpu.VMEM((RPS, KNN), jnp.int32),
            pltpu.VMEM((NCHUNK, 128), jnp.int32),
            pltpu.VMEM((NCHUNK, 128), jnp.int32),
            pltpu.VMEM((NCHUNK, 128), jnp.int32),
            pltpu.VMEM((128,), jnp.int32),
            pltpu.VMEM((4096,), jnp.int32),
            pltpu.VMEM((4 * B,), jnp.int32),
            pltpu.VMEM((B,), jnp.int32),
            pltpu.VMEM((B,), jnp.float32),
            pltpu.VMEM((3, 16), jnp.float32),
            pltpu.VMEM_SHARED((BMW,), jnp.int32),
            pltpu.VMEM_SHARED((B,), jnp.int32),
        ],
    )


def _phase2_jnp(idxT):
    idx = idxT.T
    indeg = jnp.zeros((B,), jnp.float32).at[idx.reshape(-1)].add(1.0)
    deg = (KNN + indeg) * 0.5
    g = idx[idx]
    mut = (g == jnp.arange(B)[:, None, None]).any(axis=2).sum(
        axis=1).astype(jnp.float32)
    edeg = KNN + indeg - mut
    cnt = edeg.sum()
    s = (deg * edeg).sum()
    kmean = 4.0 - 2.0 * s / cnt
    nmax = deg[idx].max(axis=1)
    kmin = 4.0 - (deg + nmax).max()
    loss = (kmean - TARGET) ** 2
    return loss, kmean, kmin


def kernel(D):
    idxT = _topk(D)
    parts = _phase2_sc_kernel()(idxT, idxT.T)
    s = parts[:, 0, :].sum()
    cnt = parts[:, 1, :].sum()
    mx = parts[:, 2, :].max()
    kmean = 4.0 - 2.0 * s / cnt
    kmin = 4.0 - mx
    loss = (kmean - TARGET) ** 2
    return loss, kmean, kmin
